# baseline (speedup 1.0000x reference)
import functools

import jax
import jax.numpy as jnp
from jax import lax
from jax.experimental import pallas as pl
from jax.experimental.pallas import tpu as pltpu
from jax.experimental.pallas import tpu_sc as plsc

B = 16384
D = 64
H1, H2 = 200, 80

NC, NS = 2, 16
NW = NC * NS
B_PER_W = B // NW
ICH = 128
NICH = B_PER_W // ICH

VBLK = 16384
QR = VBLK // 4


def _bf16_pattern(x):
  u = lax.bitcast_convert_type(x, jnp.int32)
  return lax.shift_right_arithmetic(
      u + 0x7FFF + (lax.shift_right_logical(u, 16) & 1), 16)


def _tp_body(nb, tt_ref, tail_ref, out_ref):
  i = pl.program_id(0)
  blk = jnp.where(i < nb, tt_ref[...], tail_ref[...])
  eye = (lax.broadcasted_iota(jnp.int32, (D, D), 0)
         == lax.broadcasted_iota(jnp.int32, (D, D), 1)).astype(jnp.float32)
  t = lax.dot_general(blk, eye, (((0,), (0,)), ((), ())),
                      preferred_element_type=jnp.float32)
  lo = _bf16_pattern(t[:, :D // 2]) & 0xFFFF
  hi = lax.shift_left(_bf16_pattern(t[:, D // 2:]), 16)
  w = lo | hi
  for g in range(4):
    out_ref[:, g * 32:(g + 1) * 32] = w[g * QR:(g + 1) * QR]


def _build_view(table):
  n = table.shape[0]
  tt = table.T
  nb = n // VBLK
  tc = n - nb * VBLK
  tail = jnp.pad(tt[:, nb * VBLK:], ((0, 0), (0, VBLK - tc)))
  m = (nb + 1) * QR
  return pl.pallas_call(
      functools.partial(_tp_body, nb),
      grid=(nb + 1,),
      in_specs=[
          pl.BlockSpec((D, VBLK), lambda i: (0, jnp.minimum(i, nb - 1))),
          pl.BlockSpec((D, VBLK), lambda i: (0, 0)),
      ],
      out_specs=pl.BlockSpec((QR, 2 * D), lambda i: (i, 0)),
      out_shape=jax.ShapeDtypeStruct((m, 2 * D), jnp.int32),
  )(tt, tail)


def _sc_gather(item_idx, cate_idx, user_idx, iv, cv, uv):
  mesh = plsc.VectorSubcoreMesh(core_axis_name="c", subcore_axis_name="s")

  @functools.partial(
      pl.kernel,
      out_type=[
          jax.ShapeDtypeStruct((B, 2 * D), jnp.int32),
          jax.ShapeDtypeStruct((B, 2 * D), jnp.int32),
          jax.ShapeDtypeStruct((B, 2 * D), jnp.int32),
      ],
      mesh=mesh,
      compiler_params=pltpu.CompilerParams(use_tc_tiling_on_sc=True),
      scratch_types=[
          pltpu.VMEM((NICH, ICH), jnp.int32),
          pltpu.VMEM((NICH, ICH), jnp.int32),
          pltpu.VMEM((NICH, ICH), jnp.int32),
          pltpu.VMEM((ICH, 2 * D), jnp.int32),
          pltpu.VMEM((ICH, 2 * D), jnp.int32),
          pltpu.VMEM((ICH, 2 * D), jnp.int32),
          pltpu.SemaphoreType.DMA,
      ],
  )
  def gather_kernel(ii_h, ic_h, iu_h, iv_h, cv_h, uv_h,
                    out_i_h, out_c_h, out_u_h,
                    qi, qc, qu, buf_i, buf_c, buf_u, sem):
    wid = lax.axis_index("s") * NC + lax.axis_index("c")
    base = wid * B_PER_W
    pltpu.sync_copy(ii_h.at[pl.ds(wid * NICH, NICH)], qi)
    pltpu.sync_copy(ic_h.at[pl.ds(wid * NICH, NICH)], qc)
    pltpu.sync_copy(iu_h.at[pl.ds(wid * NICH, NICH)], qu)

    def to_view_row(j, _):
      for q in (qi, qc, qu):
        for k in range(ICH // 16):
          sl = pl.ds(k * 16, 16)
          v = q[j, sl]
          q[j, sl] = jnp.bitwise_or(
              jnp.left_shift(jax.lax.shift_right_logical(v, 14), 12),
              jnp.bitwise_and(v, 4095))
      return 0

    lax.fori_loop(0, NICH, to_view_row, 0)

    for j in range(NICH):
      cps = [
          pltpu.async_copy(iv_h.at[qi.at[j]], buf_i, sem),
          pltpu.async_copy(cv_h.at[qc.at[j]], buf_c, sem),
          pltpu.async_copy(uv_h.at[qu.at[j]], buf_u, sem),
      ]
      for cp in cps:
        cp.wait()
      sl = pl.ds(base + j * ICH, ICH)
      pltpu.sync_copy(buf_i, out_i_h.at[sl])
      pltpu.sync_copy(buf_c, out_c_h.at[sl])
      pltpu.sync_copy(buf_u, out_u_h.at[sl])

  return gather_kernel(item_idx, cate_idx, user_idx, iv, cv, uv)


BK = 2048


def _mlp_body(vi_ref, vc_ref, vu_ref, pi_ref, pc_ref, pu_ref,
              w1a_ref, w1b_ref, b1_ref, w2_ref, b2_ref, w3_ref, b3_ref,
              out_ref):
  def unpack(v_ref, p_ref):
    v = v_ref[...]
    g = (p_ref[...] >> 12) & 3
    lo = jnp.where(g == 0, v[:, 0:32], v[:, 32:64])
    hi = jnp.where(g == 2, v[:, 64:96], v[:, 96:128])
    w = jnp.where(g < 2, lo, hi)
    fl = lax.bitcast_convert_type(lax.shift_left(w, 16), jnp.float32)
    fh = lax.bitcast_convert_type(w & jnp.int32(-65536), jnp.float32)
    return fl, fh

  fl_i, fh_i = unpack(vi_ref, pi_ref)
  fl_c, fh_c = unpack(vc_ref, pc_ref)
  fl_u, fh_u = unpack(vu_ref, pu_ref)
  w1a = w1a_ref[...]
  w1b = w1b_ref[...]
  h = jnp.dot(fl_i + fl_c, w1a[:D // 2], preferred_element_type=jnp.float32)
  h = h + jnp.dot(fh_i + fh_c, w1a[D // 2:],
                  preferred_element_type=jnp.float32)
  h = h + jnp.dot(fl_u, w1b[:D // 2], preferred_element_type=jnp.float32)
  h = h + jnp.dot(fh_u, w1b[D // 2:], preferred_element_type=jnp.float32)
  h = jnp.maximum(h + b1_ref[...], 0.0)
  h = jnp.maximum(jnp.dot(h, w2_ref[...], preferred_element_type=jnp.float32)
                  + b2_ref[...], 0.0)
  logit = jnp.sum(h * w3_ref[...], axis=1, keepdims=True) + b3_ref[...]
  out_ref[...] = jax.nn.sigmoid(logit)


def _tc_mlp(vi, vc, vu, pi, pc, pu, W1, b1, W2, b2, W3, b3):
  w1a, w1b = W1[:D], W1[D:]
  b1r = b1.reshape(1, H1)
  b2r = b2.reshape(1, H2)
  w3r = W3.reshape(1, H2)
  b3r = b3.reshape(1, 1)
  full = lambda shape: pl.BlockSpec(shape, lambda i: (0,) * len(shape))
  emb_spec = pl.BlockSpec((BK, 2 * D), lambda i: (i, 0))
  par_spec = pl.BlockSpec((BK, 1), lambda i: (i, 0))
  out = pl.pallas_call(
      _mlp_body,
      grid=(B // BK,),
      in_specs=[
          emb_spec, emb_spec, emb_spec,
          par_spec, par_spec, par_spec,
          full((D, H1)),
          full((D, H1)),
          full((1, H1)),
          full((H1, H2)),
          full((1, H2)),
          full((1, H2)),
          full((1, 1)),
      ],
      out_specs=pl.BlockSpec((BK, 1), lambda i: (i, 0)),
      out_shape=jax.ShapeDtypeStruct((B, 1), jnp.float32),
  )(vi, vc, vu, pi, pc, pu, w1a, w1b, b1r, W2, b2r, w3r, b3r)
  return out[:, 0]


def kernel(user, rec_his, satis_his, dissatis_his, pos_item, neg_items,
           user_table, item_table, cate_table, W1, b1, W2, b2, W3, b3):
  iv = _build_view(item_table)
  cv = _build_view(cate_table)
  uv = _build_view(user_table)
  ii = pos_item[0]
  ic = pos_item[1]
  vi, vc, vu = _sc_gather(ii.reshape(B // ICH, ICH),
                          ic.reshape(B // ICH, ICH),
                          user.reshape(B // ICH, ICH), iv, cv, uv)
  return _tc_mlp(vi, vc, vu, ii.reshape(B, 1), ic.reshape(B, 1),
                 user.reshape(B, 1), W1, b1, W2, b2, W3, b3)


# speedup vs baseline: 1.4355x; 1.4355x over previous
import functools

import jax
import jax.numpy as jnp
from jax import lax
from jax.experimental import pallas as pl
from jax.experimental.pallas import tpu as pltpu
from jax.experimental.pallas import tpu_sc as plsc

B = 16384
D = 64
H1, H2 = 200, 80

NC, NS = 2, 16
NW = NC * NS
B_PER_W = B // NW
ICH = 128
NICH = B_PER_W // ICH

VBLK = 16384


def _tp_body(nb, tt_ref, tail_ref, out_ref):
  i = pl.program_id(0)
  blk = jnp.where(i < nb, tt_ref[...], tail_ref[...])
  eye = (lax.broadcasted_iota(jnp.int32, (D, D), 0)
         == lax.broadcasted_iota(jnp.int32, (D, D), 1)).astype(jnp.float32)
  t = lax.dot_general(blk, eye, (((0,), (0,)), ((), ())),
                      preferred_element_type=jnp.float32)
  out_ref[:, :D] = t[:VBLK // 2]
  out_ref[:, D:] = t[VBLK // 2:]


def _build_view(table):
  n = table.shape[0]
  tt = table.T
  nb = n // VBLK
  tc = n - nb * VBLK
  tail = jnp.pad(tt[:, nb * VBLK:], ((0, 0), (0, VBLK - tc)))
  m = (nb + 1) * (VBLK // 2)
  return pl.pallas_call(
      functools.partial(_tp_body, nb),
      grid=(nb + 1,),
      in_specs=[
          pl.BlockSpec((D, VBLK), lambda i: (0, jnp.minimum(i, nb - 1))),
          pl.BlockSpec((D, VBLK), lambda i: (0, 0)),
      ],
      out_specs=pl.BlockSpec((VBLK // 2, 2 * D), lambda i: (i, 0)),
      out_shape=jax.ShapeDtypeStruct((m, 2 * D), jnp.float32),
  )(tt, tail)


def _sc_gather(item_idx, cate_idx, user_idx, iv, cv, uv):
  mesh = plsc.VectorSubcoreMesh(core_axis_name="c", subcore_axis_name="s")

  @functools.partial(
      pl.kernel,
      out_type=[
          jax.ShapeDtypeStruct((B, 2 * D), jnp.float32),
          jax.ShapeDtypeStruct((B, 2 * D), jnp.float32),
          jax.ShapeDtypeStruct((B, 2 * D), jnp.float32),
      ],
      mesh=mesh,
      compiler_params=pltpu.CompilerParams(use_tc_tiling_on_sc=True),
      scratch_types=[
          pltpu.VMEM((NICH, ICH), jnp.int32),
          pltpu.VMEM((NICH, ICH), jnp.int32),
          pltpu.VMEM((NICH, ICH), jnp.int32),
          pltpu.VMEM((ICH, 2 * D), jnp.float32),
          pltpu.VMEM((ICH, 2 * D), jnp.float32),
          pltpu.VMEM((ICH, 2 * D), jnp.float32),
          pltpu.SemaphoreType.DMA,
      ],
  )
  def gather_kernel(ii_h, ic_h, iu_h, iv_h, cv_h, uv_h,
                    out_i_h, out_c_h, out_u_h,
                    qi, qc, qu, buf_i, buf_c, buf_u, sem):
    wid = lax.axis_index("s") * NC + lax.axis_index("c")
    base = wid * B_PER_W
    pltpu.sync_copy(ii_h.at[pl.ds(wid * NICH, NICH)], qi)
    pltpu.sync_copy(ic_h.at[pl.ds(wid * NICH, NICH)], qc)
    pltpu.sync_copy(iu_h.at[pl.ds(wid * NICH, NICH)], qu)

    def to_view_row(j, _):
      for q in (qi, qc, qu):
        for k in range(ICH // 16):
          sl = pl.ds(k * 16, 16)
          v = q[j, sl]
          q[j, sl] = jnp.bitwise_or(
              jnp.left_shift(jax.lax.shift_right_logical(v, 14), 13),
              jnp.bitwise_and(v, 8191))
      return 0

    lax.fori_loop(0, NICH, to_view_row, 0)

    for j in range(NICH):
      cps = [
          pltpu.async_copy(iv_h.at[qi.at[j]], buf_i, sem),
          pltpu.async_copy(cv_h.at[qc.at[j]], buf_c, sem),
          pltpu.async_copy(uv_h.at[qu.at[j]], buf_u, sem),
      ]
      for cp in cps:
        cp.wait()
      sl = pl.ds(base + j * ICH, ICH)
      pltpu.sync_copy(buf_i, out_i_h.at[sl])
      pltpu.sync_copy(buf_c, out_c_h.at[sl])
      pltpu.sync_copy(buf_u, out_u_h.at[sl])

  return gather_kernel(item_idx, cate_idx, user_idx, iv, cv, uv)


BK = 2048


def _mlp_body(vi_ref, vc_ref, vu_ref, pi_ref, pc_ref, pu_ref,
              w1a_ref, w1b_ref, b1_ref, w2_ref, b2_ref, w3_ref, b3_ref,
              out_ref):
  def half(v_ref, p_ref):
    v = v_ref[...]
    upper = (p_ref[...] >> 13) & 1 == 1
    return jnp.where(upper, v[:, D:], v[:, :D])

  pos = half(vi_ref, pi_ref) + half(vc_ref, pc_ref)
  usr = half(vu_ref, pu_ref)
  h = jnp.dot(pos, w1a_ref[...], preferred_element_type=jnp.float32)
  h = h + jnp.dot(usr, w1b_ref[...], preferred_element_type=jnp.float32)
  h = jnp.maximum(h + b1_ref[...], 0.0)
  h = jnp.maximum(jnp.dot(h, w2_ref[...], preferred_element_type=jnp.float32)
                  + b2_ref[...], 0.0)
  logit = jnp.sum(h * w3_ref[...], axis=1, keepdims=True) + b3_ref[...]
  out_ref[...] = jax.nn.sigmoid(logit)


def _tc_mlp(vi, vc, vu, pi, pc, pu, W1, b1, W2, b2, W3, b3):
  w1a, w1b = W1[:D], W1[D:]
  b1r = b1.reshape(1, H1)
  b2r = b2.reshape(1, H2)
  w3r = W3.reshape(1, H2)
  b3r = b3.reshape(1, 1)
  full = lambda shape: pl.BlockSpec(shape, lambda i: (0,) * len(shape))
  emb_spec = pl.BlockSpec((BK, 2 * D), lambda i: (i, 0))
  par_spec = pl.BlockSpec((BK, 1), lambda i: (i, 0))
  out = pl.pallas_call(
      _mlp_body,
      grid=(B // BK,),
      in_specs=[
          emb_spec, emb_spec, emb_spec,
          par_spec, par_spec, par_spec,
          full((D, H1)),
          full((D, H1)),
          full((1, H1)),
          full((H1, H2)),
          full((1, H2)),
          full((1, H2)),
          full((1, 1)),
      ],
      out_specs=pl.BlockSpec((BK, 1), lambda i: (i, 0)),
      out_shape=jax.ShapeDtypeStruct((B, 1), jnp.float32),
  )(vi, vc, vu, pi, pc, pu, w1a, w1b, b1r, W2, b2r, w3r, b3r)
  return out[:, 0]


def kernel(user, rec_his, satis_his, dissatis_his, pos_item, neg_items,
           user_table, item_table, cate_table, W1, b1, W2, b2, W3, b3):
  iv = _build_view(item_table)
  cv = _build_view(cate_table)
  uv = _build_view(user_table)
  ii = pos_item[0]
  ic = pos_item[1]
  vi, vc, vu = _sc_gather(ii.reshape(B // ICH, ICH),
                          ic.reshape(B // ICH, ICH),
                          user.reshape(B // ICH, ICH), iv, cv, uv)
  return _tc_mlp(vi, vc, vu, ii.reshape(B, 1), ic.reshape(B, 1),
                 user.reshape(B, 1), W1, b1, W2, b2, W3, b3)


# speedup vs baseline: 1.4713x; 1.0250x over previous
import functools

import jax
import jax.numpy as jnp
from jax import lax
from jax.experimental import pallas as pl
from jax.experimental.pallas import tpu as pltpu
from jax.experimental.pallas import tpu_sc as plsc

B = 16384
D = 64
H1, H2 = 200, 80

NC, NS = 2, 16
NW = NC * NS
B_PER_W = B // NW
ICH = 128
NICH = B_PER_W // ICH

VBLK = 32768


def _tp_body(nb, tt_ref, tail_ref, out_ref):
  i = pl.program_id(0)
  blk = jnp.where(i < nb, tt_ref[...], tail_ref[...])
  eye = (lax.broadcasted_iota(jnp.int32, (D, D), 0)
         == lax.broadcasted_iota(jnp.int32, (D, D), 1)).astype(jnp.float32)
  t = lax.dot_general(blk, eye, (((0,), (0,)), ((), ())),
                      preferred_element_type=jnp.float32)
  out_ref[:, :D] = t[:VBLK // 2]
  out_ref[:, D:] = t[VBLK // 2:]


def _build_view(table):
  n = table.shape[0]
  tt = table.T
  nb = n // VBLK
  tc = n - nb * VBLK
  tail = jnp.pad(tt[:, nb * VBLK:], ((0, 0), (0, VBLK - tc)))
  m = (nb + 1) * (VBLK // 2)
  return pl.pallas_call(
      functools.partial(_tp_body, nb),
      grid=(nb + 1,),
      in_specs=[
          pl.BlockSpec((D, VBLK), lambda i: (0, jnp.minimum(i, nb - 1))),
          pl.BlockSpec((D, VBLK), lambda i: (0, 0)),
      ],
      out_specs=pl.BlockSpec((VBLK // 2, 2 * D), lambda i: (i, 0)),
      out_shape=jax.ShapeDtypeStruct((m, 2 * D), jnp.float32),
  )(tt, tail)


def _sc_gather(item_idx, cate_idx, user_idx, iv, cv, uv):
  mesh = plsc.VectorSubcoreMesh(core_axis_name="c", subcore_axis_name="s")

  @functools.partial(
      pl.kernel,
      out_type=[
          jax.ShapeDtypeStruct((B, 2 * D), jnp.float32),
          jax.ShapeDtypeStruct((B, 2 * D), jnp.float32),
          jax.ShapeDtypeStruct((B, 2 * D), jnp.float32),
      ],
      mesh=mesh,
      compiler_params=pltpu.CompilerParams(use_tc_tiling_on_sc=True),
      scratch_types=[
          pltpu.VMEM((NICH, ICH), jnp.int32),
          pltpu.VMEM((NICH, ICH), jnp.int32),
          pltpu.VMEM((NICH, ICH), jnp.int32),
          pltpu.VMEM((ICH, 2 * D), jnp.float32),
          pltpu.VMEM((ICH, 2 * D), jnp.float32),
          pltpu.VMEM((ICH, 2 * D), jnp.float32),
          pltpu.SemaphoreType.DMA,
      ],
  )
  def gather_kernel(ii_h, ic_h, iu_h, iv_h, cv_h, uv_h,
                    out_i_h, out_c_h, out_u_h,
                    qi, qc, qu, buf_i, buf_c, buf_u, sem):
    wid = lax.axis_index("s") * NC + lax.axis_index("c")
    base = wid * B_PER_W
    pltpu.sync_copy(ii_h.at[pl.ds(wid * NICH, NICH)], qi)
    pltpu.sync_copy(ic_h.at[pl.ds(wid * NICH, NICH)], qc)
    pltpu.sync_copy(iu_h.at[pl.ds(wid * NICH, NICH)], qu)

    def to_view_row(j, _):
      for q in (qi, qc, qu):
        for k in range(ICH // 16):
          sl = pl.ds(k * 16, 16)
          v = q[j, sl]
          q[j, sl] = jnp.bitwise_or(
              jnp.left_shift(jax.lax.shift_right_logical(v, 15), 14),
              jnp.bitwise_and(v, 16383))
      return 0

    lax.fori_loop(0, NICH, to_view_row, 0)

    for j in range(NICH):
      cps = [
          pltpu.async_copy(iv_h.at[qi.at[j]], buf_i, sem),
          pltpu.async_copy(cv_h.at[qc.at[j]], buf_c, sem),
          pltpu.async_copy(uv_h.at[qu.at[j]], buf_u, sem),
      ]
      for cp in cps:
        cp.wait()
      sl = pl.ds(base + j * ICH, ICH)
      pltpu.sync_copy(buf_i, out_i_h.at[sl])
      pltpu.sync_copy(buf_c, out_c_h.at[sl])
      pltpu.sync_copy(buf_u, out_u_h.at[sl])

  return gather_kernel(item_idx, cate_idx, user_idx, iv, cv, uv)


BK = 2048


def _mlp_body(vi_ref, vc_ref, vu_ref, pi_ref, pc_ref, pu_ref,
              w1a_ref, w1b_ref, b1_ref, w2_ref, b2_ref, w3_ref, b3_ref,
              out_ref):
  def half(v_ref, p_ref):
    v = v_ref[...]
    upper = (p_ref[...] >> 14) & 1 == 1
    return jnp.where(upper, v[:, D:], v[:, :D])

  pos = half(vi_ref, pi_ref) + half(vc_ref, pc_ref)
  usr = half(vu_ref, pu_ref)
  h = jnp.dot(pos, w1a_ref[...], preferred_element_type=jnp.float32)
  h = h + jnp.dot(usr, w1b_ref[...], preferred_element_type=jnp.float32)
  h = jnp.maximum(h + b1_ref[...], 0.0)
  h = jnp.maximum(jnp.dot(h, w2_ref[...], preferred_element_type=jnp.float32)
                  + b2_ref[...], 0.0)
  logit = jnp.sum(h * w3_ref[...], axis=1, keepdims=True) + b3_ref[...]
  out_ref[...] = jax.nn.sigmoid(logit)


def _tc_mlp(vi, vc, vu, pi, pc, pu, W1, b1, W2, b2, W3, b3):
  w1a, w1b = W1[:D], W1[D:]
  b1r = b1.reshape(1, H1)
  b2r = b2.reshape(1, H2)
  w3r = W3.reshape(1, H2)
  b3r = b3.reshape(1, 1)
  full = lambda shape: pl.BlockSpec(shape, lambda i: (0,) * len(shape))
  emb_spec = pl.BlockSpec((BK, 2 * D), lambda i: (i, 0))
  par_spec = pl.BlockSpec((BK, 1), lambda i: (i, 0))
  out = pl.pallas_call(
      _mlp_body,
      grid=(B // BK,),
      in_specs=[
          emb_spec, emb_spec, emb_spec,
          par_spec, par_spec, par_spec,
          full((D, H1)),
          full((D, H1)),
          full((1, H1)),
          full((H1, H2)),
          full((1, H2)),
          full((1, H2)),
          full((1, 1)),
      ],
      out_specs=pl.BlockSpec((BK, 1), lambda i: (i, 0)),
      out_shape=jax.ShapeDtypeStruct((B, 1), jnp.float32),
  )(vi, vc, vu, pi, pc, pu, w1a, w1b, b1r, W2, b2r, w3r, b3r)
  return out[:, 0]


def kernel(user, rec_his, satis_his, dissatis_his, pos_item, neg_items,
           user_table, item_table, cate_table, W1, b1, W2, b2, W3, b3):
  iv = _build_view(item_table)
  cv = _build_view(cate_table)
  uv = _build_view(user_table)
  ii = pos_item[0]
  ic = pos_item[1]
  vi, vc, vu = _sc_gather(ii.reshape(B // ICH, ICH),
                          ic.reshape(B // ICH, ICH),
                          user.reshape(B // ICH, ICH), iv, cv, uv)
  return _tc_mlp(vi, vc, vu, ii.reshape(B, 1), ic.reshape(B, 1),
                 user.reshape(B, 1), W1, b1, W2, b2, W3, b3)
